# Initial kernel scaffold; baseline (speedup 1.0000x reference)
import jax
import jax.numpy as jnp
from jax.experimental import pallas as pl


def kernel(X, adj_edge_index, adj_values, W_embed, W_kernels, alpha):
    raise NotImplementedError("write your pallas kernel here")


# speedup vs baseline: 2.0974x; 2.0974x over previous
import functools

import jax
import jax.numpy as jnp
from jax import lax
from jax.experimental import pallas as pl
from jax.experimental.pallas import tpu as pltpu
from jax.experimental.pallas import tpu_sc as plsc

N = 10000
F = 256
OUT = 256
NH = 4
E = 160000
HALF = 128
LANES = 16

RB = 400
GRID = N // RB

NC = 2
NS = 16
EPT = E // NS
BB = 80
NBATCH = EPT // BB
R0 = 624
TAIL = N - R0 * NS
ZB = 16


def _mm_body(x_ref, w_ref, le_ref, fw_ref, g_ref, acc_ref):
    i = pl.program_id(0)
    p = lax.dot_general(x_ref[...], w_ref[...], (((1,), (0,)), ((), ())),
                        preferred_element_type=jnp.float32)
    le = p[:, :OUT]
    le_ref[...] = le
    for h in range(NH):
        base = OUT + h * OUT
        for s in range(NC):
            fw_ref[h, s] = p[:, base + s * HALF:base + (s + 1) * HALF]
    fwsum = (p[:, OUT:2 * OUT] + p[:, 2 * OUT:3 * OUT]
             + p[:, 3 * OUT:4 * OUT] + p[:, 4 * OUT:5 * OUT])
    contrib = lax.dot_general(le, fwsum, (((0,), (0,)), ((), ())),
                              preferred_element_type=jnp.float32)

    @pl.when(i == 0)
    def _():
        acc_ref[...] = jnp.zeros_like(acc_ref)

    acc_ref[...] += contrib

    @pl.when(i == pl.num_programs(0) - 1)
    def _():
        g_ref[...] = acc_ref[...]


_mm_call = pl.pallas_call(
    _mm_body,
    grid=(GRID,),
    in_specs=[
        pl.BlockSpec((RB, F), lambda i: (i, 0)),
        pl.BlockSpec((F, (NH + 1) * OUT), lambda i: (0, 0)),
    ],
    out_specs=[
        pl.BlockSpec((RB, OUT), lambda i: (i, 0)),
        pl.BlockSpec((NH, NC, RB, HALF), lambda i: (0, 0, i, 0)),
        pl.BlockSpec((OUT, OUT), lambda i: (0, 0)),
    ],
    out_shape=[
        jax.ShapeDtypeStruct((N, OUT), jnp.float32),
        jax.ShapeDtypeStruct((NH, NC, N, HALF), jnp.float32),
        jax.ShapeDtypeStruct((OUT, OUT), jnp.float32),
    ],
    scratch_shapes=[pltpu.VMEM((OUT, OUT), jnp.float32)],
)


def _sc_spmm_body(fw_hbm, rows_hbm, cols_hbm, vals_hbm, out_hbm,
                  cols_v, rows_v, idx_v, vals_v, gath_v, zbuf_v, acc_sh, sem):
    c = lax.axis_index("c")
    s = lax.axis_index("s")

    for r in range(ZB):
        for j in range(HALF // LANES):
            zbuf_v[r, pl.ds(j * LANES, LANES)] = jnp.zeros((LANES,), jnp.float32)

    def zcopy(k, carry):
        pltpu.sync_copy(zbuf_v, acc_sh.at[pl.ds(s * R0 + k * ZB, ZB)])
        return carry

    lax.fori_loop(0, R0 // ZB, zcopy, 0)

    @pl.when(s == NS - 1)
    def _():
        pltpu.sync_copy(zbuf_v, acc_sh.at[pl.ds(R0 * NS, TAIL)])

    plsc.subcore_barrier()

    for h in range(NH):
        off = (h * NC + c) * N
        off_vec = jnp.zeros((LANES,), jnp.int32) + off

        def batch(bi, carry):
            base = h * E + s * EPT + bi * BB
            pltpu.sync_copy(cols_hbm.at[pl.ds(base, BB)], cols_v)
            pltpu.sync_copy(rows_hbm.at[pl.ds(base, BB)], rows_v)
            pltpu.sync_copy(vals_hbm.at[pl.ds(base, BB)], vals_v)
            for j in range(BB // LANES):
                sl = pl.ds(j * LANES, LANES)
                idx_v[sl] = cols_v[sl] + off_vec
            pltpu.async_copy(fw_hbm.at[idx_v], gath_v, sem).wait()

            def group(g, carry2):
                vv = vals_v[pl.ds(g * LANES, LANES)]
                for l in range(LANES):
                    b = g * LANES + l
                    v = vv[l]
                    for j in range(HALF // LANES):
                        sl = pl.ds(j * LANES, LANES)
                        gath_v[b, sl] = gath_v[b, sl] * v
                return carry2

            lax.fori_loop(0, BB // LANES, group, 0)
            pltpu.sync_copy(gath_v, acc_sh.at[rows_v], add=True)
            return carry

        lax.fori_loop(0, NBATCH, batch, 0)

    plsc.subcore_barrier()
    pltpu.sync_copy(acc_sh.at[pl.ds(s * R0, R0)],
                    out_hbm.at[c, pl.ds(s * R0, R0)])

    @pl.when(s == NS - 1)
    def _():
        pltpu.sync_copy(acc_sh.at[pl.ds(R0 * NS, TAIL)],
                        out_hbm.at[c, pl.ds(R0 * NS, TAIL)])


_sc_call = functools.partial(
    pl.kernel,
    mesh=plsc.VectorSubcoreMesh(core_axis_name="c", subcore_axis_name="s"),
    out_type=jax.ShapeDtypeStruct((NC, N, HALF), jnp.float32),
    scratch_types=[
        pltpu.VMEM((BB,), jnp.int32),
        pltpu.VMEM((BB,), jnp.int32),
        pltpu.VMEM((BB,), jnp.int32),
        pltpu.VMEM((BB,), jnp.float32),
        pltpu.VMEM((BB, HALF), jnp.float32),
        pltpu.VMEM((ZB, HALF), jnp.float32),
        pltpu.VMEM_SHARED((N, HALF), jnp.float32),
        pltpu.SemaphoreType.DMA,
    ],
)(_sc_spmm_body)


def _out_body(alpha_ref, st_ref, le_ref, g_ref, o_ref):
    a = alpha_ref[0, 0]
    lg = lax.dot_general(le_ref[...], g_ref[...], (((1,), (0,)), ((), ())),
                         preferred_element_type=jnp.float32)
    o_ref[:, :HALF] = jnp.maximum(st_ref[0] + a * lg[:, :HALF], 0.0)
    o_ref[:, HALF:] = jnp.maximum(st_ref[1] + a * lg[:, HALF:], 0.0)


_out_call = pl.pallas_call(
    _out_body,
    grid=(GRID,),
    in_specs=[
        pl.BlockSpec(memory_space=pltpu.SMEM),
        pl.BlockSpec((NC, RB, HALF), lambda i: (0, i, 0)),
        pl.BlockSpec((RB, OUT), lambda i: (i, 0)),
        pl.BlockSpec((OUT, OUT), lambda i: (0, 0)),
    ],
    out_specs=pl.BlockSpec((RB, OUT), lambda i: (i, 0)),
    out_shape=jax.ShapeDtypeStruct((N, OUT), jnp.float32),
)


def kernel(X, adj_edge_index, adj_values, W_embed, W_kernels, alpha):
    w_cat = jnp.concatenate(
        [W_embed] + [W_kernels[h] for h in range(NH)], axis=1)
    le, fw, g = _mm_call(X, w_cat)
    fw_flat = fw.reshape(NH * NC * N, HALF)
    rows_flat = adj_edge_index[:, 0, :].reshape(-1)
    cols_flat = adj_edge_index[:, 1, :].reshape(-1)
    vals_flat = adj_values.reshape(-1)
    structural = _sc_call(fw_flat, rows_flat, cols_flat, vals_flat)
    alpha_arr = jnp.reshape(alpha, (1, 1)).astype(jnp.float32)
    return _out_call(alpha_arr, structural, le, g)


# speedup vs baseline: 5.2962x; 2.5251x over previous
import functools

import jax
import jax.numpy as jnp
from jax import lax
from jax.experimental import pallas as pl
from jax.experimental.pallas import tpu as pltpu
from jax.experimental.pallas import tpu_sc as plsc

N = 10000
F = 256
OUT = 256
NH = 4
E = 160000
HALF = 128
LANES = 16

RB = 400
GRID = N // RB

NC = 2
NS = 16
EPT = E // NS
BB = 80
NBATCH = EPT // BB
R0 = 624
TAIL = N - R0 * NS
ZB = 16


def _mm_body(x_ref, w_ref, le_ref, fw_ref, g_ref, acc_ref):
    i = pl.program_id(0)
    p = lax.dot_general(x_ref[...], w_ref[...], (((1,), (0,)), ((), ())),
                        preferred_element_type=jnp.float32)
    le = p[:, :OUT]
    le_ref[...] = le
    for h in range(NH):
        base = OUT + h * OUT
        for s in range(NC):
            fw_ref[h, s] = p[:, base + s * HALF:base + (s + 1) * HALF]
    fwsum = (p[:, OUT:2 * OUT] + p[:, 2 * OUT:3 * OUT]
             + p[:, 3 * OUT:4 * OUT] + p[:, 4 * OUT:5 * OUT])
    contrib = lax.dot_general(le, fwsum, (((0,), (0,)), ((), ())),
                              preferred_element_type=jnp.float32)

    @pl.when(i == 0)
    def _():
        acc_ref[...] = jnp.zeros_like(acc_ref)

    acc_ref[...] += contrib

    @pl.when(i == pl.num_programs(0) - 1)
    def _():
        g_ref[...] = acc_ref[...]


_mm_call = pl.pallas_call(
    _mm_body,
    grid=(GRID,),
    in_specs=[
        pl.BlockSpec((RB, F), lambda i: (i, 0)),
        pl.BlockSpec((F, (NH + 1) * OUT), lambda i: (0, 0)),
    ],
    out_specs=[
        pl.BlockSpec((RB, OUT), lambda i: (i, 0)),
        pl.BlockSpec((NH, NC, RB, HALF), lambda i: (0, 0, i, 0)),
        pl.BlockSpec((OUT, OUT), lambda i: (0, 0)),
    ],
    out_shape=[
        jax.ShapeDtypeStruct((N, OUT), jnp.float32),
        jax.ShapeDtypeStruct((NH, NC, N, HALF), jnp.float32),
        jax.ShapeDtypeStruct((OUT, OUT), jnp.float32),
    ],
    scratch_shapes=[pltpu.VMEM((OUT, OUT), jnp.float32)],
)


def _sc_spmm_body(fw_hbm, rows_hbm, cols_hbm, vals_hbm, out_hbm,
                  g0, g1, cb0, cb1, vb0, vb1, ib0, ib1, rb0, rb1, zbuf_v,
                  acc_sh, gs0, gs1, ss0, ss1, ms0, ms1, rs0, rs1):
    c = lax.axis_index("c")
    s = lax.axis_index("s")
    gath = (g0, g1)
    colsb = (cb0, cb1)
    valsb = (vb0, vb1)
    idxb = (ib0, ib1)
    rbs = (rb0, rb1)
    gsem = (gs0, gs1)
    ssem = (ss0, ss1)
    msem = (ms0, ms1)
    rsem = (rs0, rs1)

    for r in range(ZB):
        for j in range(HALF // LANES):
            zbuf_v[r, pl.ds(j * LANES, LANES)] = jnp.zeros((LANES,), jnp.float32)

    def zcopy(k, carry):
        pltpu.sync_copy(zbuf_v, acc_sh.at[pl.ds(s * R0 + k * ZB, ZB)])
        return carry

    lax.fori_loop(0, R0 // ZB, zcopy, 0)

    @pl.when(s == NS - 1)
    def _():
        pltpu.sync_copy(zbuf_v, acc_sh.at[pl.ds(R0 * NS, TAIL)])

    plsc.subcore_barrier()

    def issue_meta(slot, hb, b):
        pltpu.async_copy(cols_hbm.at[pl.ds(hb + b * BB, BB)], colsb[slot],
                         msem[slot])
        pltpu.async_copy(vals_hbm.at[pl.ds(hb + b * BB, BB)], valsb[slot],
                         msem[slot])

    def wait_meta(slot, hb):
        pltpu.make_async_copy(cols_hbm.at[pl.ds(hb, BB)], colsb[slot],
                              msem[slot]).wait()
        pltpu.make_async_copy(vals_hbm.at[pl.ds(hb, BB)], valsb[slot],
                              msem[slot]).wait()

    def mkidx(slot, off_vec):
        for j in range(BB // LANES):
            sl = pl.ds(j * LANES, LANES)
            idxb[slot][sl] = colsb[slot][sl] + off_vec

    def issue_gather_rows(slot, hb, b):
        pltpu.async_copy(fw_hbm.at[idxb[slot]], gath[slot], gsem[slot])
        pltpu.async_copy(rows_hbm.at[pl.ds(hb + b * BB, BB)], rbs[slot],
                         rsem[slot])

    def wait_gather(slot):
        pltpu.make_async_copy(fw_hbm.at[idxb[slot]], gath[slot],
                              gsem[slot]).wait()

    def wait_rows(slot, hb):
        pltpu.make_async_copy(rows_hbm.at[pl.ds(hb, BB)], rbs[slot],
                              rsem[slot]).wait()

    def wait_scatter(slot):
        pltpu.make_async_copy(gath[slot], acc_sh.at[rbs[slot]],
                              ssem[slot]).wait()

    def issue_scatter(slot):
        pltpu.async_copy(gath[slot], acc_sh.at[rbs[slot]], ssem[slot],
                         add=True)

    def scale(slot):
        def grp(g, carry):
            vv = valsb[slot][pl.ds(g * LANES, LANES)]
            for l in range(LANES):
                r = g * LANES + l
                v = vv[l]
                for j in range(HALF // LANES):
                    sl = pl.ds(j * LANES, LANES)
                    gath[slot][r, sl] = gath[slot][r, sl] * v
            return carry

        lax.fori_loop(0, BB // LANES, grp, 0)

    def hop_body(h, carry):
        hb = h * E + s * EPT
        off_vec = jnp.zeros((LANES,), jnp.int32) + (h * NC + c) * N

        issue_meta(0, hb, jnp.int32(0))
        wait_meta(0, hb)
        mkidx(0, off_vec)
        issue_gather_rows(0, hb, jnp.int32(0))
        issue_meta(1, hb, jnp.int32(1))

        def step(p, k, guard_scatter, issue_next_meta):
            b = 2 * p + k
            nk = 1 - k
            wait_gather(k)
            wait_meta(nk, hb)
            mkidx(nk, off_vec)
            if guard_scatter:
                @pl.when(p > 0)
                def _():
                    wait_scatter(nk)
            else:
                wait_scatter(nk)
            issue_gather_rows(nk, hb, b + 1)
            scale(k)
            if issue_next_meta == "always":
                issue_meta(k, hb, b + 2)
            elif issue_next_meta == "guarded":
                @pl.when(p < NBATCH // 2 - 1)
                def _():
                    issue_meta(k, hb, b + 2)
            wait_rows(k, hb)
            issue_scatter(k)

        def pair(p, cy):
            step(p, 0, True, "always")
            step(p, 1, False, "guarded")
            return cy

        lax.fori_loop(0, NBATCH // 2, pair, 0)

        wait_gather(0)
        scale(0)
        wait_rows(0, hb)
        issue_scatter(0)
        wait_scatter(0)
        wait_scatter(1)
        return carry

    lax.fori_loop(0, NH, hop_body, 0)

    plsc.subcore_barrier()
    pltpu.sync_copy(acc_sh.at[pl.ds(s * R0, R0)],
                    out_hbm.at[c, pl.ds(s * R0, R0)])

    @pl.when(s == NS - 1)
    def _():
        pltpu.sync_copy(acc_sh.at[pl.ds(R0 * NS, TAIL)],
                        out_hbm.at[c, pl.ds(R0 * NS, TAIL)])


_sc_call = functools.partial(
    pl.kernel,
    mesh=plsc.VectorSubcoreMesh(core_axis_name="c", subcore_axis_name="s"),
    out_type=jax.ShapeDtypeStruct((NC, N, HALF), jnp.float32),
    scratch_types=[
        pltpu.VMEM((BB, HALF), jnp.float32),
        pltpu.VMEM((BB, HALF), jnp.float32),
        pltpu.VMEM((BB,), jnp.int32),
        pltpu.VMEM((BB,), jnp.int32),
        pltpu.VMEM((BB,), jnp.float32),
        pltpu.VMEM((BB,), jnp.float32),
        pltpu.VMEM((BB,), jnp.int32),
        pltpu.VMEM((BB,), jnp.int32),
        pltpu.VMEM((BB,), jnp.int32),
        pltpu.VMEM((BB,), jnp.int32),
        pltpu.VMEM((ZB, HALF), jnp.float32),
        pltpu.VMEM_SHARED((N, HALF), jnp.float32),
        pltpu.SemaphoreType.DMA,
        pltpu.SemaphoreType.DMA,
        pltpu.SemaphoreType.DMA,
        pltpu.SemaphoreType.DMA,
        pltpu.SemaphoreType.DMA,
        pltpu.SemaphoreType.DMA,
        pltpu.SemaphoreType.DMA,
        pltpu.SemaphoreType.DMA,
    ],
)(_sc_spmm_body)


def _out_body(alpha_ref, st_ref, le_ref, g_ref, o_ref):
    a = alpha_ref[0, 0]
    lg = lax.dot_general(le_ref[...], g_ref[...], (((1,), (0,)), ((), ())),
                         preferred_element_type=jnp.float32)
    o_ref[:, :HALF] = jnp.maximum(st_ref[0] + a * lg[:, :HALF], 0.0)
    o_ref[:, HALF:] = jnp.maximum(st_ref[1] + a * lg[:, HALF:], 0.0)


_out_call = pl.pallas_call(
    _out_body,
    grid=(GRID,),
    in_specs=[
        pl.BlockSpec(memory_space=pltpu.SMEM),
        pl.BlockSpec((NC, RB, HALF), lambda i: (0, i, 0)),
        pl.BlockSpec((RB, OUT), lambda i: (i, 0)),
        pl.BlockSpec((OUT, OUT), lambda i: (0, 0)),
    ],
    out_specs=pl.BlockSpec((RB, OUT), lambda i: (i, 0)),
    out_shape=jax.ShapeDtypeStruct((N, OUT), jnp.float32),
)


def kernel(X, adj_edge_index, adj_values, W_embed, W_kernels, alpha):
    w_cat = jnp.concatenate(
        [W_embed] + [W_kernels[h] for h in range(NH)], axis=1)
    le, fw, g = _mm_call(X, w_cat)
    fw_flat = fw.reshape(NH * NC * N, HALF)
    rows_flat = adj_edge_index[:, 0, :].reshape(-1)
    cols_flat = adj_edge_index[:, 1, :].reshape(-1)
    vals_flat = adj_values.reshape(-1)
    structural = _sc_call(fw_flat, rows_flat, cols_flat, vals_flat)
    alpha_arr = jnp.reshape(alpha, (1, 1)).astype(jnp.float32)
    return _out_call(alpha_arr, structural, le, g)
